# baseline (speedup 1.0000x reference)
import functools

import jax
import jax.numpy as jnp
from jax import lax
from jax.experimental import pallas as pl
from jax.experimental.pallas import tpu as pltpu
from jax.experimental.pallas import tpu_sc as plsc

NUM_EMBEDDINGS = 8192
EMBEDDING_DIM = 32
TOKENS_PER_BLOCK = 256
N_TOKENS = 8192

_SC_CORES = 2
_SC_SUBCORES = 16
_N_WORKERS = _SC_CORES * _SC_SUBCORES
_B_PER_W = N_TOKENS // _N_WORKERS


_LANES = 128
_N_CHUNKS = NUM_EMBEDDINGS // _LANES


def _vq_argmin_block(z_ref, w_ref, idx_ref, loss_ref, w2_ref, wn_ref):
    i = pl.program_id(0)

    @pl.when(i == 0)
    def _():
        w = w_ref[...]
        w2_ref[...] = jnp.sum(w * w, axis=1).reshape(1, NUM_EMBEDDINGS)
        wn_ref[...] = -2.0 * w
        loss_ref[...] = jnp.zeros((1, 1), jnp.float32)

    zb = z_ref[...]
    z2 = jnp.sum(zb * zb, axis=1, keepdims=True)

    def _mm_chunk(c):
        return lax.dot_general(
            zb, wn_ref[c * _LANES:(c + 1) * _LANES, :],
            dimension_numbers=(((1,), (1,)), ((), ())),
            preferred_element_type=jnp.float32,
        )

    m = (z2 + w2_ref[0:1, 0:_LANES]) + _mm_chunk(0)
    a = jnp.zeros(m.shape, jnp.float32)
    for c in range(1, _N_CHUNKS):
        lo, hi = c * _LANES, (c + 1) * _LANES
        dc = (z2 + w2_ref[0:1, lo:hi]) + _mm_chunk(c)
        a = jnp.where(dc < m, jnp.float32(c), a)
        m = jnp.minimum(m, dc)
    minval = jnp.min(m, axis=1, keepdims=True)
    lane = lax.broadcasted_iota(jnp.int32, m.shape, 1).astype(jnp.float32)
    cand = jnp.where(m == minval, a * _LANES + lane, jnp.float32(NUM_EMBEDDINGS))
    idx = jnp.min(cand, axis=1).astype(jnp.int32)
    idx_ref[0, 0, :] = idx
    loss_ref[...] += jnp.sum(minval).reshape(1, 1)


_PAD_DIM = 128


def _sc_gather_body(table_hbm, idx_hbm, out_hbm, idx_v, rows_v, sem):
    wid = lax.axis_index("s") * _SC_CORES + lax.axis_index("c")
    base = wid * _B_PER_W
    pltpu.sync_copy(idx_hbm.at[pl.ds(base, _B_PER_W)], idx_v)
    pltpu.async_copy(table_hbm.at[idx_v], rows_v, sem).wait()
    pltpu.sync_copy(rows_v, out_hbm.at[pl.ds(base, _B_PER_W)])


@functools.cache
def _make_sc_gather():
    return pl.kernel(
        _sc_gather_body,
        out_type=jax.ShapeDtypeStruct((N_TOKENS, _PAD_DIM), jnp.float32),
        mesh=plsc.VectorSubcoreMesh(core_axis_name="c", subcore_axis_name="s"),
        scratch_types=[
            pltpu.VMEM((_B_PER_W,), jnp.int32),
            pltpu.VMEM((_B_PER_W, _PAD_DIM), jnp.float32),
            pltpu.SemaphoreType.DMA,
        ],
    )


def _sc_gather(table_pad, idx_flat):
    return _make_sc_gather()(table_pad, idx_flat)


def kernel(z, embed_w):
    B, S, D = z.shape
    K = embed_w.shape[0]
    n = B * S
    zf = z.reshape(n, D)
    nblk = n // TOKENS_PER_BLOCK
    idx3, lacc = pl.pallas_call(
        _vq_argmin_block,
        grid=(nblk,),
        in_specs=[
            pl.BlockSpec((TOKENS_PER_BLOCK, D), lambda i: (i, 0)),
            pl.BlockSpec((K, D), lambda i: (0, 0)),
        ],
        out_specs=[
            pl.BlockSpec((1, 1, TOKENS_PER_BLOCK), lambda i: (i, 0, 0)),
            pl.BlockSpec((1, 1), lambda i: (0, 0)),
        ],
        out_shape=[
            jax.ShapeDtypeStruct((nblk, 1, TOKENS_PER_BLOCK), jnp.int32),
            jax.ShapeDtypeStruct((1, 1), jnp.float32),
        ],
        scratch_shapes=[
            pltpu.VMEM((1, K), jnp.float32),
            pltpu.VMEM((K, D), jnp.float32),
        ],
    )(zf, embed_w)
    idx_flat = idx3.reshape(n)
    table_pad = jnp.pad(embed_w, ((0, 0), (0, _PAD_DIM - D)))
    quant = _sc_gather(table_pad, idx_flat)[:, :D]
    mse = lacc[0, 0] / jnp.float32(n * D)
    loss = mse * 0.25 + mse
    return (loss, quant.reshape(B, S, D), idx_flat.reshape(B, S))


# speedup vs baseline: 1.0635x; 1.0635x over previous
import functools

import jax
import jax.numpy as jnp
from jax import lax
from jax.experimental import pallas as pl
from jax.experimental.pallas import tpu as pltpu
from jax.experimental.pallas import tpu_sc as plsc

NUM_EMBEDDINGS = 8192
EMBEDDING_DIM = 32
TOKENS_PER_BLOCK = 1024
N_TOKENS = 8192

_SC_CORES = 2
_SC_SUBCORES = 16
_N_WORKERS = _SC_CORES * _SC_SUBCORES
_B_PER_W = N_TOKENS // _N_WORKERS


_LANES = 128
_N_CHUNKS = NUM_EMBEDDINGS // _LANES


def _vq_argmin_block(z_ref, w_ref, idx_ref, loss_ref, w2_ref, wn_ref):
    i = pl.program_id(0)

    @pl.when(i == 0)
    def _():
        w = w_ref[...]
        w2_ref[...] = jnp.sum(w * w, axis=1).reshape(1, NUM_EMBEDDINGS)
        wn_ref[...] = -2.0 * w
        loss_ref[...] = jnp.zeros((1, 1), jnp.float32)

    zb = z_ref[...]
    z2 = jnp.sum(zb * zb, axis=1, keepdims=True)

    mmneg = lax.dot_general(
        zb, wn_ref[...],
        dimension_numbers=(((1,), (1,)), ((), ())),
        preferred_element_type=jnp.float32,
    )
    m = (z2 + w2_ref[0:1, 0:_LANES]) + mmneg[:, 0:_LANES]
    a = jnp.zeros(m.shape, jnp.float32)
    for c in range(1, _N_CHUNKS):
        lo, hi = c * _LANES, (c + 1) * _LANES
        dc = (z2 + w2_ref[0:1, lo:hi]) + mmneg[:, lo:hi]
        a = jnp.where(dc < m, jnp.float32(c), a)
        m = jnp.minimum(m, dc)
    minval = jnp.min(m, axis=1, keepdims=True)
    lane = lax.broadcasted_iota(jnp.int32, m.shape, 1).astype(jnp.float32)
    cand = jnp.where(m == minval, a * _LANES + lane, jnp.float32(NUM_EMBEDDINGS))
    idx = jnp.min(cand, axis=1).astype(jnp.int32)
    idx_ref[0, 0, :] = idx
    loss_ref[...] += jnp.sum(minval).reshape(1, 1)


_PAD_DIM = 128


def _sc_gather_body(table_hbm, idx_hbm, out_hbm, idx_v, rows_v, sem):
    wid = lax.axis_index("s") * _SC_CORES + lax.axis_index("c")
    base = wid * _B_PER_W
    pltpu.sync_copy(idx_hbm.at[pl.ds(base, _B_PER_W)], idx_v)
    pltpu.async_copy(table_hbm.at[idx_v], rows_v, sem).wait()
    pltpu.sync_copy(rows_v, out_hbm.at[pl.ds(base, _B_PER_W)])


@functools.cache
def _make_sc_gather():
    return pl.kernel(
        _sc_gather_body,
        out_type=jax.ShapeDtypeStruct((N_TOKENS, _PAD_DIM), jnp.float32),
        mesh=plsc.VectorSubcoreMesh(core_axis_name="c", subcore_axis_name="s"),
        scratch_types=[
            pltpu.VMEM((_B_PER_W,), jnp.int32),
            pltpu.VMEM((_B_PER_W, _PAD_DIM), jnp.float32),
            pltpu.SemaphoreType.DMA,
        ],
    )


def _sc_gather(table_pad, idx_flat):
    return _make_sc_gather()(table_pad, idx_flat)


def kernel(z, embed_w):
    B, S, D = z.shape
    K = embed_w.shape[0]
    n = B * S
    zf = z.reshape(n, D)
    nblk = n // TOKENS_PER_BLOCK
    idx3, lacc = pl.pallas_call(
        _vq_argmin_block,
        grid=(nblk,),
        in_specs=[
            pl.BlockSpec((TOKENS_PER_BLOCK, D), lambda i: (i, 0)),
            pl.BlockSpec((K, D), lambda i: (0, 0)),
        ],
        out_specs=[
            pl.BlockSpec((1, 1, TOKENS_PER_BLOCK), lambda i: (i, 0, 0)),
            pl.BlockSpec((1, 1), lambda i: (0, 0)),
        ],
        out_shape=[
            jax.ShapeDtypeStruct((nblk, 1, TOKENS_PER_BLOCK), jnp.int32),
            jax.ShapeDtypeStruct((1, 1), jnp.float32),
        ],
        scratch_shapes=[
            pltpu.VMEM((1, K), jnp.float32),
            pltpu.VMEM((K, D), jnp.float32),
        ],
    )(zf, embed_w)
    idx_flat = idx3.reshape(n)
    table_pad = jnp.pad(embed_w, ((0, 0), (0, _PAD_DIM - D)))
    quant = _sc_gather(table_pad, idx_flat)[:, :D]
    mse = lacc[0, 0] / jnp.float32(n * D)
    loss = mse * 0.25 + mse
    return (loss, quant.reshape(B, S, D), idx_flat.reshape(B, S))
